# Initial kernel scaffold; baseline (speedup 1.0000x reference)
import jax
import jax.numpy as jnp
from jax.experimental import pallas as pl


def kernel(x, theta):
    raise NotImplementedError("write your pallas kernel here")


# speedup vs baseline: 1176.9236x; 1176.9236x over previous
import functools
import math

import jax
import jax.numpy as jnp
from jax import lax
from jax.experimental import pallas as pl
from jax.experimental.pallas import tpu as pltpu
from jax.experimental.pallas import tpu_sc as plsc

DEPTH_L = 18
NUM_LEAVES = 2 ** DEPTH_L
NUM_NODES_K = NUM_LEAVES - 1
BATCH = 2000000
SCALE = float(NUM_LEAVES)
BONUS = DEPTH_L * math.log(2.0)

NC, NS, LANES = 2, 16, 16
NW = NC * NS

ALOG_LEN = 2 * NUM_NODES_K
ALOG_ROWS = 4104
ALOG_PAD = ALOG_ROWS * 128

TPB = NUM_LEAVES // NW
_OFFC = [2 ** (DEPTH_L - m) - 2 for m in range(DEPTH_L)]
_SPAN = [max(TPB >> m, 1) for m in range(DEPTH_L)]
_LEN = [(-(-(s + 8) // 16)) * 16 for s in _SPAN]
_BASE = [sum(_LEN[:m]) for m in range(DEPTH_L)]
STAGE_TOTAL = sum(_LEN)

CHUNK = 4000
NCHUNKS = BATCH // CHUNK
MAX_ITERS = -(-NCHUNKS // NW)

_MESH = plsc.VectorSubcoreMesh(
    core_axis_name="c", subcore_axis_name="s", num_cores=NC, num_subcores=NS)


def _log_body(th_ref, out_ref):
    v = jnp.log(th_ref[...] + 1e-20)
    r = lax.broadcasted_iota(jnp.int32, v.shape, 0)
    q = lax.broadcasted_iota(jnp.int32, v.shape, 1)
    out_ref[...] = v + jnp.where((r == 0) & (q < 2), BONUS, 0.0)


_log_call = pl.pallas_call(
    _log_body,
    out_shape=jax.ShapeDtypeStruct((ALOG_ROWS, 128), jnp.float32),
)


@functools.partial(
    pl.kernel,
    out_type=jax.ShapeDtypeStruct((NUM_LEAVES,), jnp.float32),
    mesh=_MESH,
    compiler_params=pltpu.CompilerParams(needs_layout_passes=False),
    scratch_types=[
        pltpu.VMEM((STAGE_TOTAL,), jnp.float32),
        pltpu.VMEM((TPB,), jnp.float32),
        pltpu.SemaphoreType.DMA,
    ],
)
def _build_table(alog_hbm, s_hbm, stage_v, out_v, sem):
    wid = lax.axis_index("s") * NC + lax.axis_index("c")
    c0 = wid * TPB
    descs = []
    adjs = []
    for m in range(DEPTH_L):
        c0s = jnp.right_shift(c0, m)
        off = _OFFC[m] + c0s
        off_al = pl.multiple_of(jnp.bitwise_and(off, jnp.int32(-8)), 8)
        descs.append(pltpu.async_copy(
            alog_hbm.at[pl.ds(off_al, _LEN[m])],
            stage_v.at[pl.ds(_BASE[m], _LEN[m])], sem))
        adjs.append(_BASE[m] + (off - off_al) - c0s)
    for d in descs:
        d.wait()

    iota = lax.iota(jnp.int32, LANES)

    def body(t, carry):
        c_vec = c0 + t * LANES + iota
        acc = plsc.load_gather(stage_v, [c_vec + adjs[0]])
        for m in range(1, DEPTH_L):
            acc = acc + plsc.load_gather(
                stage_v, [jnp.right_shift(c_vec, m) + adjs[m]])
        out_v[pl.ds(t * LANES, LANES)] = acc
        return carry

    lax.fori_loop(0, TPB // LANES, body, 0)
    pltpu.sync_copy(out_v, s_hbm.at[pl.ds(c0, TPB)])


@functools.partial(
    pl.kernel,
    out_type=jax.ShapeDtypeStruct((BATCH,), jnp.float32),
    mesh=_MESH,
    scratch_types=[
        pltpu.VMEM((CHUNK,), jnp.float32),
        pltpu.VMEM((CHUNK,), jnp.int32),
        pltpu.VMEM((CHUNK,), jnp.float32),
        pltpu.SemaphoreType.DMA,
    ],
)
def _gather_leaves(x_hbm, s_hbm, out_hbm, x_v, idx_v, r_v, sem):
    wid = lax.axis_index("s") * NC + lax.axis_index("c")
    for k in range(MAX_ITERS):
        cid = k * NW + wid

        @pl.when(cid < NCHUNKS)
        def _():
            base = cid * CHUNK
            pltpu.sync_copy(x_hbm.at[pl.ds(base, CHUNK)], x_v)

            def body(t, carry):
                xv = x_v[pl.ds(t * LANES, LANES)]
                ci = (xv * SCALE).astype(jnp.int32)
                ci = jnp.minimum(jnp.maximum(ci, 0), NUM_LEAVES - 1)
                idx_v[pl.ds(t * LANES, LANES)] = ci
                return carry

            lax.fori_loop(0, CHUNK // LANES, body, 0)
            pltpu.async_copy(s_hbm.at[idx_v], r_v, sem).wait()
            pltpu.sync_copy(r_v, out_hbm.at[pl.ds(base, CHUNK)])


def kernel(x, theta):
    th_flat = theta.reshape(-1)
    th_pad = jnp.concatenate(
        [th_flat, jnp.ones((ALOG_PAD - ALOG_LEN,), th_flat.dtype)])
    alog = _log_call(th_pad.reshape(ALOG_ROWS, 128)).reshape(-1)
    s_table = _build_table(alog)
    return _gather_leaves(x, s_table)


# speedup vs baseline: 1230.4203x; 1.0455x over previous
import functools
import math

import jax
import jax.numpy as jnp
from jax import lax
from jax.experimental import pallas as pl
from jax.experimental.pallas import tpu as pltpu
from jax.experimental.pallas import tpu_sc as plsc

DEPTH_L = 18
NUM_LEAVES = 2 ** DEPTH_L
NUM_NODES_K = NUM_LEAVES - 1
BATCH = 2000000
SCALE = float(NUM_LEAVES)
BONUS = DEPTH_L * math.log(2.0)
TH_LEN = 2 * NUM_NODES_K

NC, NS, LANES = 2, 16, 16
NW = NC * NS

_LOG_C = (0.043428907822139526, -0.4048671744191854, 1.5939013634991297,
          -3.49249427987935, 5.046876044975941, -2.786812953867443)
_LN2 = math.log(2.0)

TPB = NUM_LEAVES // NW
_OFFC = [2 ** (DEPTH_L - m) - 2 for m in range(DEPTH_L)]
_SPAN = [max(TPB >> m, 1) for m in range(DEPTH_L)]
_ALLOC = [(-(-(s + 8) // 16)) * 16 for s in _SPAN]
_BASE = [sum(_ALLOC[:m]) for m in range(DEPTH_L)]
STAGE_TOTAL = sum(_ALLOC)
_LEN0 = _SPAN[0] + 6

CHUNK = 4000
NCHUNKS = BATCH // CHUNK
MAX_ITERS = -(-NCHUNKS // NW)

_MESH = plsc.VectorSubcoreMesh(
    core_axis_name="c", subcore_axis_name="s", num_cores=NC, num_subcores=NS)
_PARAMS = pltpu.CompilerParams(needs_layout_passes=False)


def _vlog(v):
    bits = plsc.bitcast(v, jnp.int32)
    e = jnp.right_shift(bits, 23) - 127
    mant = plsc.bitcast(
        jnp.bitwise_or(jnp.bitwise_and(bits, 0x007FFFFF), 0x3F800000),
        jnp.float32)
    acc = mant * _LOG_C[0] + _LOG_C[1]
    for coef in _LOG_C[2:]:
        acc = acc * mant + coef
    return (acc + e.astype(jnp.float32)) * _LN2


@functools.partial(
    pl.kernel,
    out_type=jax.ShapeDtypeStruct((NUM_LEAVES,), jnp.float32),
    mesh=_MESH,
    compiler_params=_PARAMS,
    scratch_types=[
        pltpu.VMEM((STAGE_TOTAL,), jnp.float32),
        pltpu.VMEM((TPB,), jnp.float32),
        pltpu.SemaphoreType.DMA,
    ],
)
def _build_table(th_hbm, s_hbm, stage_v, out_v, sem):
    wid = lax.axis_index("s") * NC + lax.axis_index("c")
    c0 = wid * TPB

    descs = []
    adjs = [None] * DEPTH_L
    off0 = pl.multiple_of(c0 + (_OFFC[0] - 6), 8)
    descs.append(pltpu.async_copy(
        th_hbm.at[pl.ds(off0, _LEN0)], stage_v.at[pl.ds(_BASE[0], _LEN0)],
        sem))
    adjs[0] = _BASE[0] + 6 - c0
    for m in range(1, DEPTH_L):
        c0s = jnp.right_shift(c0, m)
        off = _OFFC[m] + c0s
        off_al = pl.multiple_of(jnp.bitwise_and(off, jnp.int32(-8)), 8)
        descs.append(pltpu.async_copy(
            th_hbm.at[pl.ds(off_al, _ALLOC[m])],
            stage_v.at[pl.ds(_BASE[m], _ALLOC[m])], sem))
        adjs[m] = _BASE[m] + (off - off_al) - c0s
    for d in descs:
        d.wait()

    def logbody(j, carry):
        sl = stage_v[pl.ds(j * LANES, LANES)]
        stage_v[pl.ds(j * LANES, LANES)] = _vlog(sl + 1e-20)
        return carry

    lax.fori_loop(0, STAGE_TOTAL // LANES, logbody, 0)

    iota = lax.iota(jnp.int32, LANES)

    def body(t, carry):
        c_vec = c0 + t * LANES + iota
        acc = plsc.load_gather(stage_v, [c_vec + adjs[0]]) + BONUS
        for m in range(1, DEPTH_L):
            acc = acc + plsc.load_gather(
                stage_v, [jnp.right_shift(c_vec, m) + adjs[m]])
        out_v[pl.ds(t * LANES, LANES)] = acc
        return carry

    lax.fori_loop(0, TPB // LANES, body, 0)
    pltpu.sync_copy(out_v, s_hbm.at[pl.ds(c0, TPB)])


@functools.partial(
    pl.kernel,
    out_type=jax.ShapeDtypeStruct((BATCH,), jnp.float32),
    mesh=_MESH,
    compiler_params=_PARAMS,
    scratch_types=[
        pltpu.VMEM((CHUNK,), jnp.float32),
        pltpu.VMEM((CHUNK,), jnp.float32),
        pltpu.VMEM((CHUNK,), jnp.int32),
        pltpu.VMEM((CHUNK,), jnp.int32),
        pltpu.VMEM((CHUNK,), jnp.float32),
        pltpu.VMEM((CHUNK,), jnp.float32),
        pltpu.SemaphoreType.DMA,
        pltpu.SemaphoreType.DMA,
        pltpu.SemaphoreType.DMA,
        pltpu.SemaphoreType.DMA,
        pltpu.SemaphoreType.DMA,
    ],
)
def _gather_leaves(x_hbm, s_hbm, out_hbm,
                   x0, x1, i0, i1, r0, r1, sx0, sx1, sg, ss0, ss1):
    wid = lax.axis_index("s") * NC + lax.axis_index("c")
    xs, idxs, rs = (x0, x1), (i0, i1), (r0, r1)
    sxs, sss = (sx0, sx1), (ss0, ss1)

    def chunk_base(k):
        cid = k * NW + wid
        cid = jnp.where(cid < NCHUNKS, cid, cid - NW)
        return pl.multiple_of(cid * CHUNK, 8)

    def idx_compute(b):
        def body(t, carry):
            xv = xs[b][pl.ds(t * LANES, LANES)]
            ci = (xv * SCALE).astype(jnp.int32)
            ci = jnp.minimum(jnp.maximum(ci, 0), NUM_LEAVES - 1)
            idxs[b][pl.ds(t * LANES, LANES)] = ci
            return carry
        lax.fori_loop(0, CHUNK // LANES, body, 0)

    dx = [None, None]
    dg = [None, None]
    dst = [None, None]
    dx[0] = pltpu.async_copy(
        x_hbm.at[pl.ds(chunk_base(0), CHUNK)], xs[0], sxs[0])
    for k in range(MAX_ITERS):
        b = k & 1
        if k + 1 < MAX_ITERS:
            dx[1 - b] = pltpu.async_copy(
                x_hbm.at[pl.ds(chunk_base(k + 1), CHUNK)], xs[1 - b],
                sxs[1 - b])
        dx[b].wait()
        idx_compute(b)
        if k >= 1:
            dg[1 - b].wait()
            dst[1 - b] = pltpu.async_copy(
                rs[1 - b], out_hbm.at[pl.ds(chunk_base(k - 1), CHUNK)],
                sss[1 - b])
        if k >= 2:
            dst[b].wait()
        dg[b] = pltpu.async_copy(s_hbm.at[idxs[b]], rs[b], sg)
    bl = (MAX_ITERS - 1) & 1
    dg[bl].wait()
    dst[bl] = pltpu.async_copy(
        rs[bl], out_hbm.at[pl.ds(chunk_base(MAX_ITERS - 1), CHUNK)], sss[bl])
    dst[1 - bl].wait()
    dst[bl].wait()


def kernel(x, theta):
    s_table = _build_table(theta.reshape(-1))
    return _gather_leaves(x, s_table)
